# Initial kernel scaffold; baseline (speedup 1.0000x reference)
import jax
import jax.numpy as jnp
from jax.experimental import pallas as pl


def kernel(tokens, table):
    raise NotImplementedError("write your pallas kernel here")


# speedup vs baseline: 7.1421x; 7.1421x over previous
import functools
import math

import jax
import jax.numpy as jnp
from jax import lax
from jax.experimental import pallas as pl
from jax.experimental.pallas import tpu as pltpu
from jax.experimental.pallas import tpu_sc as plsc

_VOCAB = 100000
_EMB = 128
_SCALE = math.sqrt(float(_EMB))

_B = 4096 * 200
_NW = 32
_BPW = _B // _NW
_C = 128
_NCHUNK = _BPW // _C
_K = 4
_NGROUP = _NCHUNK // _K


def _scale_table(table):
    def body(t_ref, o_ref):
        o_ref[...] = t_ref[...] * _SCALE

    return pl.pallas_call(
        body,
        grid=(100,),
        in_specs=[pl.BlockSpec((_VOCAB // 100, _EMB), lambda i: (i, 0))],
        out_specs=pl.BlockSpec((_VOCAB // 100, _EMB), lambda i: (i, 0)),
        out_shape=jax.ShapeDtypeStruct((_VOCAB, _EMB), jnp.float32),
    )(table)


_mesh = plsc.VectorSubcoreMesh(core_axis_name="c", subcore_axis_name="s")


@functools.partial(
    pl.kernel,
    mesh=_mesh,
    out_type=jax.ShapeDtypeStruct((_B, _EMB), jnp.float32),
    scratch_types=[
        pltpu.VMEM((_NCHUNK, _C), jnp.int32),
        pltpu.VMEM((_K, _C, _EMB), jnp.float32),
        pltpu.SemaphoreType.DMA,
        pltpu.SemaphoreType.DMA,
    ],
)
def _gather(tokens_hbm, table_hbm, out_hbm, idx_v, rows_v, gsem, ssem):
    cid = lax.axis_index("c")
    sid = lax.axis_index("s")
    wid = sid * 2 + cid
    base = wid * _BPW

    pltpu.sync_copy(tokens_hbm.at[wid], idx_v)

    def group(i, _):
        g0 = i * _K
        gcps = [
            pltpu.async_copy(table_hbm.at[idx_v.at[g0 + b]], rows_v.at[b], gsem)
            for b in range(_K)
        ]
        for cp in gcps:
            cp.wait()
        scps = [
            pltpu.async_copy(
                rows_v.at[b], out_hbm.at[pl.ds(base + (g0 + b) * _C, _C)], ssem
            )
            for b in range(_K)
        ]
        for cp in scps:
            cp.wait()
        return _

    lax.fori_loop(0, _NGROUP, group, 0)


def kernel(tokens, table):
    table_scaled = _scale_table(table)
    tok = tokens.reshape(_NW, _NCHUNK, _C).astype(jnp.int32)
    out = _gather(tok, table_scaled)
    return out.reshape(tokens.shape[0], tokens.shape[1], _EMB)


# speedup vs baseline: 7.5530x; 1.0575x over previous
import functools
import math

import jax
import jax.numpy as jnp
from jax import lax
from jax.experimental import pallas as pl
from jax.experimental.pallas import tpu as pltpu
from jax.experimental.pallas import tpu_sc as plsc

_VOCAB = 100000
_EMB = 128
_SCALE = math.sqrt(float(_EMB))

_B = 4096 * 200
_NW = 32
_BPW = _B // _NW
_C = 128
_NCHUNK = _BPW // _C
_NBUF = 5


def _scale_table(table):
    def body(t_ref, o_ref):
        o_ref[...] = t_ref[...] * _SCALE

    return pl.pallas_call(
        body,
        grid=(100,),
        in_specs=[pl.BlockSpec((_VOCAB // 100, _EMB), lambda i: (i, 0))],
        out_specs=pl.BlockSpec((_VOCAB // 100, _EMB), lambda i: (i, 0)),
        out_shape=jax.ShapeDtypeStruct((_VOCAB, _EMB), jnp.float32),
    )(table)


_mesh = plsc.VectorSubcoreMesh(core_axis_name="c", subcore_axis_name="s")


@functools.partial(
    pl.kernel,
    mesh=_mesh,
    out_type=jax.ShapeDtypeStruct((_B, _EMB), jnp.float32),
    scratch_types=[
        pltpu.VMEM((_NCHUNK, _C), jnp.int32),
        pltpu.VMEM((_NBUF, _C, _EMB), jnp.float32),
        pltpu.SemaphoreType.DMA,
    ],
)
def _gather(tokens_hbm, table_hbm, out_hbm, idx_v, rows_v, gsem):
    cid = lax.axis_index("c")
    sid = lax.axis_index("s")
    wid = sid * 2 + cid
    base = wid * _BPW

    pltpu.sync_copy(tokens_hbm.at[wid], idx_v)

    def g_copy(g, b):
        return pltpu.make_async_copy(
            table_hbm.at[idx_v.at[g]], rows_v.at[b], gsem
        )

    def emit(g, b):
        g_copy(g, b).wait()
        pltpu.sync_copy(rows_v.at[b], out_hbm.at[pl.ds(base + g * _C, _C)])

    for b in range(_NBUF):
        g_copy(b, b).start()

    def body(i, _):
        g0 = i * _NBUF
        for b in range(_NBUF):
            emit(g0 + b, b)
            g_copy(g0 + b + _NBUF, b).start()
        return _

    lax.fori_loop(0, _NCHUNK // _NBUF - 1, body, 0)

    for b in range(_NBUF):
        emit(_NCHUNK - _NBUF + b, b)


def kernel(tokens, table):
    table_scaled = _scale_table(table)
    tok = tokens.reshape(_NW, _NCHUNK, _C).astype(jnp.int32)
    out = _gather(tok, table_scaled)
    return out.reshape(tokens.shape[0], tokens.shape[1], _EMB)


# speedup vs baseline: 9.2187x; 1.2205x over previous
import functools
import math

import jax
import jax.numpy as jnp
from jax import lax
from jax.experimental import pallas as pl
from jax.experimental.pallas import tpu as pltpu
from jax.experimental.pallas import tpu_sc as plsc

_VOCAB = 100000
_EMB = 128
_SCALE = math.sqrt(float(_EMB))

_B = 4096 * 200
_NW = 32
_BPW = _B // _NW
_C = 128
_NCHUNK = _BPW // _C
_NBUF = 5
_GA = 3

_mesh = plsc.VectorSubcoreMesh(core_axis_name="c", subcore_axis_name="s")


@functools.partial(
    pl.kernel,
    mesh=_mesh,
    out_type=jax.ShapeDtypeStruct((_B, _EMB), jnp.float32),
    scratch_types=[
        pltpu.VMEM((_NCHUNK, _C), jnp.int32),
        pltpu.VMEM((_NBUF, _C, _EMB), jnp.float32),
        pltpu.SemaphoreType.DMA,
        pltpu.SemaphoreType.DMA,
    ],
)
def _gather(tokens_hbm, table_hbm, out_hbm, idx_v, rows_v, gsem, ssem):
    cid = lax.axis_index("c")
    sid = lax.axis_index("s")
    wid = sid * 2 + cid
    base = wid * _BPW

    pltpu.sync_copy(tokens_hbm.at[wid], idx_v)

    def g_copy(g, b):
        return pltpu.make_async_copy(
            table_hbm.at[idx_v.at[g]], rows_v.at[b], gsem
        )

    def s_copy(g, b):
        return pltpu.make_async_copy(
            rows_v.at[b], out_hbm.at[pl.ds(base + g * _C, _C)], ssem
        )

    def scale(b):
        def sbody(r, carry):
            for c in range(_EMB // 16):
                sl = pl.ds(c * 16, 16)
                rows_v[b, r, sl] = rows_v[b, r, sl] * _SCALE
            return carry

        lax.fori_loop(0, _C, sbody, 0)

    def chunk(g, b, wait_s, issue_g):
        g_copy(g, b).wait()
        scale(b)
        s_copy(g, b).start()
        if wait_s:
            s_copy(g - (_NBUF - _GA), (b - (_NBUF - _GA)) % _NBUF).wait()
        if issue_g:
            g_copy(g + _GA, (b + _GA) % _NBUF).start()

    for g in range(_GA):
        g_copy(g, g).start()

    for b in range(_NBUF):
        chunk(b, b, wait_s=(b >= _NBUF - _GA), issue_g=True)

    def body(i, carry):
        g0 = i * _NBUF
        for b in range(_NBUF):
            chunk(g0 + b, b, wait_s=True, issue_g=True)
        return carry

    lax.fori_loop(1, _NCHUNK // _NBUF - 1, body, 0)

    g0 = _NCHUNK - _NBUF
    for b in range(_NBUF):
        chunk(g0 + b, b, wait_s=True, issue_g=(b + _GA < _NBUF))

    for g in range(_NCHUNK - (_NBUF - _GA), _NCHUNK):
        s_copy(g, g % _NBUF).wait()


def kernel(tokens, table):
    tok = tokens.reshape(_NW, _NCHUNK, _C).astype(jnp.int32)
    out = _gather(tok, table)
    return out.reshape(tokens.shape[0], tokens.shape[1], _EMB)
